# baseline (speedup 1.0000x reference)
import functools

import jax
import jax.numpy as jnp
from jax import lax
from jax.experimental import pallas as pl
from jax.experimental.pallas import tpu as pltpu
from jax.experimental.pallas import tpu_sc as plsc

_NEGS = 5
_SCALING = 0.1
_WIN = 128
_TBK = 8192

_PAR = pltpu.CompilerParams(dimension_semantics=("parallel",))


def _tc_transpose_fused(at, bt):
    D, V = at.shape
    grid = (V + _TBK - 1) // _TBK

    def body(a_r, b_r, o_r):
        x = jnp.concatenate([a_r[...], b_r[...]], axis=0)
        o_r[...] = jnp.transpose(x, (1, 0)).astype(jnp.bfloat16)

    return pl.pallas_call(
        body,
        grid=(grid,),
        in_specs=[pl.BlockSpec((D, _TBK), lambda i: (0, i)),
                  pl.BlockSpec((D, _TBK), lambda i: (0, i))],
        out_specs=pl.BlockSpec((_TBK, 2 * D), lambda i: (i, 0)),
        out_shape=jax.ShapeDtypeStruct((grid * _TBK, 2 * D), jnp.bfloat16),
        compiler_params=_PAR,
    )(at, bt)


def _sc_gather_out(mu_idx, rho_idx, nz_idx, tab):
    n = mu_idx.shape[1]
    n_nz = nz_idx.shape[1]
    D = tab.shape[1]
    bf16 = jnp.bfloat16
    mesh = plsc.VectorSubcoreMesh(core_axis_name="c", subcore_axis_name="s")
    out_type = [jax.ShapeDtypeStruct((n, D), bf16),
                jax.ShapeDtypeStruct((n, D), bf16),
                jax.ShapeDtypeStruct((n_nz, D), bf16)]
    ispec = pl.BlockSpec((1, _WIN), lambda i: (0, i))
    ospec = pl.BlockSpec((_WIN, D), lambda i: (i, 0))

    @functools.partial(
        pl.kernel, out_type=out_type, mesh=mesh,
        compiler_params=pltpu.CompilerParams(use_tc_tiling_on_sc=False))
    def gk(mu_idx_h, rho_idx_h, nz_idx_h, t_h, mu_h, rho_h, nz_h):
        def body(i_v, o_v):
            pltpu.sync_copy(t_h.at[i_v.at[0]], o_v)

        pltpu.emit_pipeline(
            body, grid=(n // _WIN,),
            in_specs=[ispec], out_specs=[ospec],
            core_axis_name=("c", "s"), dimension_semantics=(pltpu.PARALLEL,),
        )(mu_idx_h, mu_h)
        pltpu.emit_pipeline(
            body, grid=(n // _WIN,),
            in_specs=[ispec], out_specs=[ospec],
            core_axis_name=("c", "s"), dimension_semantics=(pltpu.PARALLEL,),
        )(rho_idx_h, rho_h)
        pltpu.emit_pipeline(
            body, grid=(n_nz // _WIN,),
            in_specs=[ispec], out_specs=[ospec],
            core_axis_name=("c", "s"), dimension_semantics=(pltpu.PARALLEL,),
        )(nz_idx_h, nz_h)

    return gk(mu_idx, rho_idx, nz_idx, tab)


def _sc_gather_in(mu_idx, rho_idx, tab):
    n = mu_idx.shape[1]
    D = tab.shape[1]
    bf16 = jnp.bfloat16
    mesh = plsc.VectorSubcoreMesh(core_axis_name="c", subcore_axis_name="s")
    out_type = [jax.ShapeDtypeStruct((n, D), bf16),
                jax.ShapeDtypeStruct((n, D), bf16)]
    ispec = pl.BlockSpec((1, _WIN), lambda i: (0, i))
    ospec = pl.BlockSpec((_WIN, D), lambda i: (i, 0))

    @functools.partial(
        pl.kernel, out_type=out_type, mesh=mesh,
        compiler_params=pltpu.CompilerParams(use_tc_tiling_on_sc=False))
    def gk(mu_idx_h, rho_idx_h, t_h, mu_h, rho_h):
        def body(i_v, o_v):
            pltpu.sync_copy(t_h.at[i_v.at[0]], o_v)

        pltpu.emit_pipeline(
            body, grid=(n // _WIN,),
            in_specs=[ispec], out_specs=[ospec],
            core_axis_name=("c", "s"), dimension_semantics=(pltpu.PARALLEL,),
        )(mu_idx_h, mu_h)
        pltpu.emit_pipeline(
            body, grid=(n // _WIN,),
            in_specs=[ispec], out_specs=[ospec],
            core_axis_name=("c", "s"), dimension_semantics=(pltpu.PARALLEL,),
        )(rho_idx_h, rho_h)

    return gk(mu_idx, rho_idx, tab)


def _tc_math(mu_in, rho_in, eps_in, covf, covw, wT, bvec,
             mu_p, rho_p, eps_p, noise_p, B, W, D):
    GB = 128
    nblocks = B // GB
    GP = GB * W // 2
    f32 = jnp.float32
    hi = lax.Precision.HIGHEST
    halfw = W // 2

    def body(mu_in_r, rho_in_r, eps_in_r, cov_r, covw_r, wT_r, b_r,
             mu_p_r, rho_p_r, eps_p_r, noise_r, kl_r, lik_r):
        mu_in = mu_in_r[...].astype(f32)
        rho_in = rho_in_r[...].astype(f32)
        eps_in = eps_in_r[...]
        cov = cov_r[...]
        covw = covw_r[...]
        wT = wT_r[...]
        bb = b_r[...]

        y = covw[0:1, :] + cov * (covw[1:2, :] - covw[0:1, :])
        sig_in = jnp.log(jnp.exp(rho_in) + 1.0)
        h = (jnp.dot(mu_in, wT[0:D, :], precision=hi, preferred_element_type=f32)
             + jnp.dot(y, wT[D:2 * D, :], precision=hi, preferred_element_type=f32)
             + bb)
        w_in = jnp.tanh(h) + _SCALING * sig_in * eps_in
        post_in = -0.5 * jnp.sum(eps_in * eps_in) - jnp.sum(jnp.log(sig_in))
        wsq = w_in * w_in
        prior_in = jnp.sum(jnp.log(0.5 * jnp.exp(-wsq / 2.0)
                                   + 0.5 * jnp.exp(-wsq / 0.08)))
        kl = W * (post_in - prior_in)

        mo = mu_p_r[...].astype(f32)
        ro = rho_p_r[...].astype(f32)
        ep = eps_p_r[...]
        sig_o = jnp.log(jnp.exp(ro) + 1.0)
        w_o = mo + _SCALING * sig_o * ep
        post_out = -0.5 * jnp.sum(ep * ep) - jnp.sum(jnp.log(sig_o))
        wsq_o = w_o * w_o
        prior_out = jnp.sum(jnp.log(0.5 * jnp.exp(-wsq_o / 2.0)
                                    + 0.5 * jnp.exp(-wsq_o / 0.08)))
        kl += post_out - prior_out

        rowi = lax.broadcasted_iota(jnp.int32, (GP, GB), 0) // halfw
        colj = lax.broadcasted_iota(jnp.int32, (GP, GB), 1)
        sel = (rowi == colj).astype(f32)
        wsel = jnp.dot(sel, w_in, precision=hi, preferred_element_type=f32)
        wp = jnp.concatenate([wsel, wsel], axis=1)
        prodt = wp * w_o
        cols = [jnp.sum(prodt[:, 0:D], axis=1, keepdims=True),
                jnp.sum(prodt[:, D:2 * D], axis=1, keepdims=True)]

        nz = noise_r[...].astype(f32)
        half = _NEGS * D
        for j in range(_NEGS):
            pair = jnp.concatenate([nz[:, j * D:(j + 1) * D],
                                    nz[:, half + j * D:half + (j + 1) * D]],
                                   axis=1)
            prodn = wp * pair
            cols.append(-jnp.sum(prodn[:, 0:D], axis=1, keepdims=True))
            cols.append(-jnp.sum(prodn[:, D:2 * D], axis=1, keepdims=True))

        smat = jnp.concatenate(cols, axis=1)
        lg = jnp.log(jax.nn.sigmoid(smat))
        lik = (jnp.sum(lg[:, 0:2])
               + jnp.sum(lg[:, 2:2 + 2 * _NEGS]) / _NEGS)

        kl_r[...] = kl.reshape(1, 1, 1)
        lik_r[...] = lik.reshape(1, 1, 1)

    part_spec = pl.BlockSpec((1, 1, 1), lambda i: (i, 0, 0))
    kl_parts, lik_parts = pl.pallas_call(
        body,
        grid=(nblocks,),
        in_specs=[
            pl.BlockSpec((GB, D), lambda i: (i, 0)),
            pl.BlockSpec((GB, D), lambda i: (i, 0)),
            pl.BlockSpec((GB, D), lambda i: (i, 0)),
            pl.BlockSpec((GB, 1), lambda i: (i, 0)),
            pl.BlockSpec((2, D), lambda i: (0, 0)),
            pl.BlockSpec((2 * D, D), lambda i: (0, 0)),
            pl.BlockSpec((1, D), lambda i: (0, 0)),
            pl.BlockSpec((GP, 2 * D), lambda i: (i, 0)),
            pl.BlockSpec((GP, 2 * D), lambda i: (i, 0)),
            pl.BlockSpec((GP, 2 * D), lambda i: (i, 0)),
            pl.BlockSpec((GP, 2 * _NEGS * D), lambda i: (i, 0)),
        ],
        out_specs=[part_spec, part_spec],
        out_shape=[jax.ShapeDtypeStruct((nblocks, 1, 1), f32)] * 2,
        compiler_params=_PAR,
    )(mu_in, rho_in, eps_in, covf, covw, wT, bvec,
      mu_p, rho_p, eps_p, noise_p)
    return kl_parts, lik_parts


def kernel(inputs, outputs, covars, wt, batch_num, in_embed_w, out_embed_w,
           in_rho_w, out_rho_w, covariates_w, linear_w, linear_b):
    B, W = outputs.shape
    V, D = in_embed_w.shape

    key = jax.random.key(42)
    k1, k2, k3 = jax.random.split(key, 3)
    eps_in = jax.random.normal(k1, (B, D), jnp.float32)
    eps_p = jax.random.normal(k2, (B * W // 2, 2 * D), jnp.float32)
    noise_idx = jax.random.randint(k3, (B * W, _NEGS), 0, V)

    tab_out = _tc_transpose_fused(out_embed_w.T, out_rho_w.T)
    tab_in = _tc_transpose_fused(in_embed_w.T, in_rho_w.T)
    V2 = 2 * tab_out.shape[0]
    tab_out64 = tab_out.reshape(V2, D)
    tab_in64 = tab_in.reshape(V2, D)

    o2 = 2 * outputs.astype(jnp.int32).reshape(1, B * W)
    nz2 = 2 * noise_idx.astype(jnp.int32).reshape(1, B * W * _NEGS)
    i2 = 2 * inputs.astype(jnp.int32).reshape(1, B)

    mu_out_d, rho_out_d, noise_d = _sc_gather_out(o2, o2 + 1, nz2, tab_out64)
    mu_in_d, rho_in_d = _sc_gather_in(i2, i2 + 1, tab_in64)

    mu_p = mu_out_d.reshape(B * W // 2, 2 * D)
    rho_p = rho_out_d.reshape(B * W // 2, 2 * D)
    noise_p = noise_d.reshape(B * W // 2, 2 * _NEGS * D)

    covf = covars.astype(jnp.float32).reshape(B, 1)
    wT = linear_w.T
    bvec = linear_b.reshape(1, D)

    kl_parts, lik_parts = _tc_math(mu_in_d, rho_in_d, eps_in, covf,
                                   covariates_w, wT, bvec, mu_p, rho_p,
                                   eps_p, noise_p, B, W, D)
    loss = (wt[0] * jnp.sum(kl_parts) - jnp.sum(lik_parts)) / (B * W)
    return loss


# speedup vs baseline: 2.1369x; 2.1369x over previous
import functools

import jax
import jax.numpy as jnp
from jax import lax
from jax.experimental import pallas as pl
from jax.experimental.pallas import tpu as pltpu
from jax.experimental.pallas import tpu_sc as plsc

_NEGS = 5
_SCALING = 0.1
_WIN = 128
_TBK = 8192

_PAR = pltpu.CompilerParams(dimension_semantics=("parallel",))


def _tc_transpose_fused(at, bt):
    D, V = at.shape
    grid = (V + _TBK - 1) // _TBK

    def body(a_r, b_r, o_r):
        x = jnp.concatenate([a_r[...], b_r[...]], axis=0)
        o_r[...] = jnp.transpose(x, (1, 0))

    return pl.pallas_call(
        body,
        grid=(grid,),
        in_specs=[pl.BlockSpec((D, _TBK), lambda i: (0, i)),
                  pl.BlockSpec((D, _TBK), lambda i: (0, i))],
        out_specs=pl.BlockSpec((_TBK, 2 * D), lambda i: (i, 0)),
        out_shape=jax.ShapeDtypeStruct((grid * _TBK, 2 * D), jnp.float32),
        compiler_params=_PAR,
    )(at, bt)


def _sc_gather_out(mu_idx, rho_idx, nz_idx, tab):
    n = mu_idx.shape[1]
    n_nz = nz_idx.shape[1]
    D = tab.shape[1]
    f32 = jnp.float32
    mesh = plsc.VectorSubcoreMesh(core_axis_name="c", subcore_axis_name="s")
    out_type = [jax.ShapeDtypeStruct((n, D), f32),
                jax.ShapeDtypeStruct((n, D), f32),
                jax.ShapeDtypeStruct((n_nz, D), f32)]
    ispec = pl.BlockSpec((1, _WIN), lambda i: (0, i))
    ospec = pl.BlockSpec((_WIN, D), lambda i: (i, 0))

    @functools.partial(
        pl.kernel, out_type=out_type, mesh=mesh,
        compiler_params=pltpu.CompilerParams(use_tc_tiling_on_sc=False))
    def gk(mu_idx_h, rho_idx_h, nz_idx_h, t_h, mu_h, rho_h, nz_h):
        def body(i_v, o_v):
            pltpu.sync_copy(t_h.at[i_v.at[0]], o_v)

        pltpu.emit_pipeline(
            body, grid=(n // _WIN,),
            in_specs=[ispec], out_specs=[ospec],
            core_axis_name=("c", "s"), dimension_semantics=(pltpu.PARALLEL,),
        )(mu_idx_h, mu_h)
        pltpu.emit_pipeline(
            body, grid=(n // _WIN,),
            in_specs=[ispec], out_specs=[ospec],
            core_axis_name=("c", "s"), dimension_semantics=(pltpu.PARALLEL,),
        )(rho_idx_h, rho_h)
        pltpu.emit_pipeline(
            body, grid=(n_nz // _WIN,),
            in_specs=[ispec], out_specs=[ospec],
            core_axis_name=("c", "s"), dimension_semantics=(pltpu.PARALLEL,),
        )(nz_idx_h, nz_h)

    return gk(mu_idx, rho_idx, nz_idx, tab)


def _sc_gather_in(mu_idx, rho_idx, tab):
    n = mu_idx.shape[1]
    D = tab.shape[1]
    f32 = jnp.float32
    mesh = plsc.VectorSubcoreMesh(core_axis_name="c", subcore_axis_name="s")
    out_type = [jax.ShapeDtypeStruct((n, D), f32),
                jax.ShapeDtypeStruct((n, D), f32)]
    ispec = pl.BlockSpec((1, _WIN), lambda i: (0, i))
    ospec = pl.BlockSpec((_WIN, D), lambda i: (i, 0))

    @functools.partial(
        pl.kernel, out_type=out_type, mesh=mesh,
        compiler_params=pltpu.CompilerParams(use_tc_tiling_on_sc=False))
    def gk(mu_idx_h, rho_idx_h, t_h, mu_h, rho_h):
        def body(i_v, o_v):
            pltpu.sync_copy(t_h.at[i_v.at[0]], o_v)

        pltpu.emit_pipeline(
            body, grid=(n // _WIN,),
            in_specs=[ispec], out_specs=[ospec],
            core_axis_name=("c", "s"), dimension_semantics=(pltpu.PARALLEL,),
        )(mu_idx_h, mu_h)
        pltpu.emit_pipeline(
            body, grid=(n // _WIN,),
            in_specs=[ispec], out_specs=[ospec],
            core_axis_name=("c", "s"), dimension_semantics=(pltpu.PARALLEL,),
        )(rho_idx_h, rho_h)

    return gk(mu_idx, rho_idx, tab)


def _tc_math(mu_in, rho_in, eps_in, covf, covw, wT, bvec,
             mu_p, rho_p, eps_p, noise_p, B, W, D):
    GB = 128
    nblocks = B // GB
    GP = GB * W // 2
    f32 = jnp.float32
    hi = lax.Precision.HIGHEST
    halfw = W // 2

    def body(mu_in_r, rho_in_r, eps_in_r, cov_r, covw_r, wT_r, b_r,
             mu_p_r, rho_p_r, eps_p_r, noise_r, kl_r, lik_r):
        mu_in = mu_in_r[...]
        rho_in = rho_in_r[...]
        eps_in = eps_in_r[...]
        cov = cov_r[...]
        covw = covw_r[...]
        wT = wT_r[...]
        bb = b_r[...]

        y = covw[0:1, :] + cov * (covw[1:2, :] - covw[0:1, :])
        sig_in = jnp.log(jnp.exp(rho_in) + 1.0)
        h = (jnp.dot(mu_in, wT[0:D, :], precision=hi, preferred_element_type=f32)
             + jnp.dot(y, wT[D:2 * D, :], precision=hi, preferred_element_type=f32)
             + bb)
        w_in = jnp.tanh(h) + _SCALING * sig_in * eps_in
        post_in = -0.5 * jnp.sum(eps_in * eps_in) - jnp.sum(jnp.log(sig_in))
        wsq = w_in * w_in
        prior_in = jnp.sum(jnp.log(0.5 * jnp.exp(-wsq / 2.0)
                                   + 0.5 * jnp.exp(-wsq / 0.08)))
        kl = W * (post_in - prior_in)

        mo = mu_p_r[...]
        ro = rho_p_r[...]
        ep = eps_p_r[...]
        sig_o = jnp.log(jnp.exp(ro) + 1.0)
        w_o = mo + _SCALING * sig_o * ep
        post_out = -0.5 * jnp.sum(ep * ep) - jnp.sum(jnp.log(sig_o))
        wsq_o = w_o * w_o
        prior_out = jnp.sum(jnp.log(0.5 * jnp.exp(-wsq_o / 2.0)
                                    + 0.5 * jnp.exp(-wsq_o / 0.08)))
        kl += post_out - prior_out

        rowi = lax.broadcasted_iota(jnp.int32, (GP, GB), 0) // halfw
        colj = lax.broadcasted_iota(jnp.int32, (GP, GB), 1)
        sel = (rowi == colj).astype(f32)
        wsel = jnp.dot(sel, w_in, precision=hi, preferred_element_type=f32)
        wp = jnp.concatenate([wsel, wsel], axis=1)
        prodt = wp * w_o
        cols = [jnp.sum(prodt[:, 0:D], axis=1, keepdims=True),
                jnp.sum(prodt[:, D:2 * D], axis=1, keepdims=True)]

        nz = noise_r[...]
        half = _NEGS * D
        for j in range(_NEGS):
            pair = jnp.concatenate([nz[:, j * D:(j + 1) * D],
                                    nz[:, half + j * D:half + (j + 1) * D]],
                                   axis=1)
            prodn = wp * pair
            cols.append(-jnp.sum(prodn[:, 0:D], axis=1, keepdims=True))
            cols.append(-jnp.sum(prodn[:, D:2 * D], axis=1, keepdims=True))

        smat = jnp.concatenate(cols, axis=1)
        lg = jnp.log(jax.nn.sigmoid(smat))
        lik = (jnp.sum(lg[:, 0:2])
               + jnp.sum(lg[:, 2:2 + 2 * _NEGS]) / _NEGS)

        kl_r[...] = kl.reshape(1, 1, 1)
        lik_r[...] = lik.reshape(1, 1, 1)

    part_spec = pl.BlockSpec((1, 1, 1), lambda i: (i, 0, 0))
    kl_parts, lik_parts = pl.pallas_call(
        body,
        grid=(nblocks,),
        in_specs=[
            pl.BlockSpec((GB, D), lambda i: (i, 0)),
            pl.BlockSpec((GB, D), lambda i: (i, 0)),
            pl.BlockSpec((GB, D), lambda i: (i, 0)),
            pl.BlockSpec((GB, 1), lambda i: (i, 0)),
            pl.BlockSpec((2, D), lambda i: (0, 0)),
            pl.BlockSpec((2 * D, D), lambda i: (0, 0)),
            pl.BlockSpec((1, D), lambda i: (0, 0)),
            pl.BlockSpec((GP, 2 * D), lambda i: (i, 0)),
            pl.BlockSpec((GP, 2 * D), lambda i: (i, 0)),
            pl.BlockSpec((GP, 2 * D), lambda i: (i, 0)),
            pl.BlockSpec((GP, 2 * _NEGS * D), lambda i: (i, 0)),
        ],
        out_specs=[part_spec, part_spec],
        out_shape=[jax.ShapeDtypeStruct((nblocks, 1, 1), f32)] * 2,
        compiler_params=_PAR,
    )(mu_in, rho_in, eps_in, covf, covw, wT, bvec,
      mu_p, rho_p, eps_p, noise_p)
    return kl_parts, lik_parts


def kernel(inputs, outputs, covars, wt, batch_num, in_embed_w, out_embed_w,
           in_rho_w, out_rho_w, covariates_w, linear_w, linear_b):
    B, W = outputs.shape
    V, D = in_embed_w.shape

    key = jax.random.key(42)
    k1, k2, k3 = jax.random.split(key, 3)
    eps_in = jax.random.normal(k1, (B, D), jnp.float32)
    eps_p = jax.random.normal(k2, (B * W // 2, 2 * D), jnp.float32)
    noise_idx = jax.random.randint(k3, (B * W, _NEGS), 0, V)

    tab_out = _tc_transpose_fused(out_embed_w.T, out_rho_w.T)
    tab_in = _tc_transpose_fused(in_embed_w.T, in_rho_w.T)
    V2 = 2 * tab_out.shape[0]
    tab_out64 = tab_out.reshape(V2, D)
    tab_in64 = tab_in.reshape(V2, D)

    o2 = 2 * outputs.astype(jnp.int32).reshape(1, B * W)
    nz2 = 2 * noise_idx.astype(jnp.int32).reshape(1, B * W * _NEGS)
    i2 = 2 * inputs.astype(jnp.int32).reshape(1, B)

    mu_out_d, rho_out_d, noise_d = _sc_gather_out(o2, o2 + 1, nz2, tab_out64)
    mu_in_d, rho_in_d = _sc_gather_in(i2, i2 + 1, tab_in64)

    mu_p = mu_out_d.reshape(B * W // 2, 2 * D)
    rho_p = rho_out_d.reshape(B * W // 2, 2 * D)
    noise_p = noise_d.reshape(B * W // 2, 2 * _NEGS * D)

    covf = covars.astype(jnp.float32).reshape(B, 1)
    wT = linear_w.T
    bvec = linear_b.reshape(1, D)

    kl_parts, lik_parts = _tc_math(mu_in_d, rho_in_d, eps_in, covf,
                                   covariates_w, wT, bvec, mu_p, rho_p,
                                   eps_p, noise_p, B, W, D)
    loss = (wt[0] * jnp.sum(kl_parts) - jnp.sum(lik_parts)) / (B * W)
    return loss
